# baseline (speedup 1.0000x reference)
import jax
import jax.numpy as jnp
from jax.experimental import pallas as pl
from jax.experimental.pallas import tpu as pltpu
from jax.experimental.pallas import tpu_sc as plsc

_NCLS = 8
_N = 1000
_NPAD = 1024
_G = _NCLS * _NPAD
_CHUNK = 128
_TOPK = 100
_NOUT = 128
_NEG = -1e30
_BIG = 99999.0
_KEEP = 11.0
_FPAD = 32
_NW = 32
_PERW = _G // _NW


def _fiota(shape, axis):
    return jax.lax.broadcasted_iota(jnp.int32, shape, axis).astype(jnp.float32)


def _class_body(acol_ref, arow_ref, p_ref, pay_ref, scc_ref, scr_ref):
    A_c = acol_ref[0]
    A_r = arow_ref[0]
    P = p_ref[0]

    x1c, y1c, x2c, y2c = A_c[0:1, :], A_c[1:2, :], A_c[2:3, :], A_c[3:4, :]
    clsc, cfc = A_c[4:5, :], A_c[5:6, :]
    x1r, y1r, x2r, y2r = A_r[:, 0:1], A_r[:, 1:2], A_r[:, 2:3], A_r[:, 3:4]
    clsr, cfr, phr = A_r[:, 4:5], A_r[:, 5:6], A_r[:, 6:7]

    mx1 = jnp.maximum(x1r, x1c)
    my1 = jnp.maximum(y1r, y1c)
    mx2 = jnp.minimum(x2r, x2c)
    my2 = jnp.minimum(y2r, y2c)
    wid = mx2 - mx1 + 1.0
    hei = my2 - my1 + 1.0
    inter = wid * hei
    area_r = (x2r - x1r + 1.0) * (y2r - y1r + 1.0)
    area_c = (x2c - x1c + 1.0) * (y2c - y1c + 1.0)
    union = area_r + area_c - inter
    ov = jnp.where(union == 0.0, 0.0, inter / jnp.where(union == 0.0, 1.0, union))
    ov = jnp.where(wid <= 0.0, 0.0, ov)
    ov = jnp.where(hei <= 0.0, 0.0, ov)

    validr = clsr > 0.5
    validc = clsc > 0.5
    cond = (ov > 0.5) & validr & validc
    colio = _fiota((_NPAD, _NPAD), 1)
    rowio = _fiota((_NPAD, _NPAD), 0)

    jstar_r = jnp.min(jnp.where(cond, colio, _BIG), axis=1, keepdims=True)
    jstar_c = jnp.min(jnp.where(cond, rowio, _BIG), axis=0, keepdims=True)
    ovs_r = jnp.max(jnp.where(colio == jstar_r, ov, -1.0), axis=1, keepdims=True)
    ovs_c = jnp.max(jnp.where(rowio == jstar_c, ov, -1.0), axis=0, keepdims=True)
    key_r = (1.0 - ovs_r) * cfr
    key_c = (1.0 - ovs_c) * cfc
    mem_r = (jstar_r < _BIG) & (key_r != 0.0)
    mem_c = (jstar_c < _BIG) & (key_c != 0.0)

    samej = jstar_c == jstar_r
    less_cr = (key_c < key_r) | ((key_c == key_r) & (colio < rowio))
    rank_r = jnp.sum(jnp.where(samej & mem_c & less_cr, 1.0, 0.0), axis=1, keepdims=True)
    less_rc = (key_r < key_c) | ((key_r == key_c) & (rowio < colio))
    rank_c = jnp.sum(jnp.where(samej & mem_r & less_rc, 1.0, 0.0), axis=0, keepdims=True)
    sel_r = mem_r & (rank_r < _KEEP)
    sel_c = mem_c & (rank_c < _KEEP)

    W = jnp.where((rowio == jstar_c) & sel_c, 1.0, 0.0)
    Wt = jnp.where((colio == jstar_r) & sel_r, 1.0, 0.0)
    sums = jnp.dot(W, P, preferred_element_type=jnp.float32)
    d_r = jnp.sum(W, axis=1, keepdims=True)
    d_c = jnp.sum(Wt, axis=0, keepdims=True)
    den = jnp.where(d_r == 0.0, 1.0, d_r)
    zero = d_r == 0.0
    poses_o = jnp.where(zero, 0.0, sums[:, 0:12] / den)
    boxes_o = jnp.where(zero, 0.0, sums[:, 12:16] / den)

    anyv = jnp.max(jnp.where(validc, 1.0, 0.0))
    riota = _fiota((_NPAD, 1), 0)
    ciota = _fiota((1, _NPAD), 1)
    real_r = riota < float(_N)
    keep_r = (d_r > 0.0) & validr & real_r
    keep_c = (d_c > 0.0) & validc & (ciota < float(_N))
    sc_r = jnp.where(keep_r, clsr, _NEG)
    sc_c = jnp.where(keep_c, clsc, _NEG)
    ph_on = anyv == 0.0
    sc_r = jnp.where((riota == float(_N)) & ph_on, phr, sc_r)
    sc_c = jnp.where((ciota == float(_N)) & ph_on, A_c[6:7, :], sc_c)
    scr_ref[0] = sc_r
    scc_ref[0] = sc_c

    cidf = jnp.full((_NPAD, 1), pl.program_id(0), jnp.float32)
    pay_box = jnp.where(keep_r, boxes_o, -1.0)
    pay_pos = jnp.where(keep_r, poses_o, -1.0)
    pay_scr = jnp.where(sc_r == _NEG, -1.0, sc_r)
    pay_lbl = jnp.where(keep_r, cidf, -1.0)
    pay_ref[0] = jnp.concatenate(
        [pay_box, pay_pos, pay_scr, pay_lbl,
         jnp.zeros((_NPAD, _FPAD - 18), jnp.float32)], axis=1)


def _rank_body(scol_ref, srow_ref, rank_ref):
    sc = scol_ref[0:1, :]
    sr = srow_ref[...]
    k = pl.program_id(0)
    rio = _fiota((_CHUNK, 1), 0) + k.astype(jnp.float32) * float(_CHUNK)
    cio = _fiota((_CHUNK, _G), 1)
    ahead = (sc > sr) | ((sc == sr) & (cio < rio))
    rank_ref[...] = jnp.sum(jnp.where(ahead, 1.0, 0.0), axis=1, keepdims=True)


def _sc_select_body(rank_hbm, pay_hbm, out_hbm, rank_v, rows_v, sem):
    cidx = jax.lax.axis_index("c")
    sidx = jax.lax.axis_index("s")
    wid = sidx * 2 + cidx
    base = wid * _PERW
    pltpu.sync_copy(rank_hbm.at[pl.ds(base, _PERW)], rank_v)
    dump = _NOUT + wid
    lanes = jax.lax.iota(jnp.int32, 16)

    def chunk(ch, carry):
        rk = rank_v[pl.ds(ch * 16, 16)]
        slot = jnp.where(rk < _NOUT, rk, dump)
        pltpu.sync_copy(pay_hbm.at[pl.ds(base + ch * 16, 16)], rows_v)
        pltpu.async_copy(rows_v, out_hbm.at[slot], sem).wait()
        return carry + lanes[0] * 0

    jax.lax.fori_loop(0, _PERW // 16, chunk, jnp.int32(0))


def kernel(boxes3D, boxes, classification, poses, confidence):
    del boxes3D
    f32 = jnp.float32
    bx = boxes.reshape(_N, _NCLS, 4).astype(f32)
    cls2 = classification.reshape(_N, _NCLS).astype(f32)
    cf2 = confidence.reshape(_N, _NCLS).astype(f32)
    ps2 = poses.reshape(_N, _NCLS, 12).astype(f32)
    ph_score = cls2[-1, -1]

    feat = jnp.stack(
        [bx[..., 0], bx[..., 1], bx[..., 2], bx[..., 3], cls2, cf2,
         jnp.broadcast_to(ph_score, (_N, _NCLS)), jnp.zeros((_N, _NCLS), f32)],
        axis=-1)
    feat = jnp.pad(feat, ((0, _NPAD - _N), (0, 0), (0, 0)))
    a_row = feat.transpose(1, 0, 2)
    a_col = feat.transpose(1, 2, 0)
    pmat = jnp.concatenate([ps2, bx], axis=-1)
    pmat = jnp.pad(pmat, ((0, _NPAD - _N), (0, 0), (0, 0))).transpose(1, 0, 2)

    pay, sc_c, sc_r = pl.pallas_call(
        _class_body,
        grid=(_NCLS,),
        in_specs=[
            pl.BlockSpec((1, 8, _NPAD), lambda c: (c, 0, 0)),
            pl.BlockSpec((1, _NPAD, 8), lambda c: (c, 0, 0)),
            pl.BlockSpec((1, _NPAD, 16), lambda c: (c, 0, 0)),
        ],
        out_specs=[
            pl.BlockSpec((1, _NPAD, _FPAD), lambda c: (c, 0, 0)),
            pl.BlockSpec((1, 1, _NPAD), lambda c: (c, 0, 0)),
            pl.BlockSpec((1, _NPAD, 1), lambda c: (c, 0, 0)),
        ],
        out_shape=[
            jax.ShapeDtypeStruct((_NCLS, _NPAD, _FPAD), f32),
            jax.ShapeDtypeStruct((_NCLS, 1, _NPAD), f32),
            jax.ShapeDtypeStruct((_NCLS, _NPAD, 1), f32),
        ],
    )(a_col, a_row, pmat)

    s_col = sc_c.reshape(1, _G)
    s_row = sc_r.reshape(_G, 1)

    rank = pl.pallas_call(
        _rank_body,
        grid=(_G // _CHUNK,),
        in_specs=[
            pl.BlockSpec((1, _G), lambda k: (0, 0)),
            pl.BlockSpec((_CHUNK, 1), lambda k: (k, 0)),
        ],
        out_specs=pl.BlockSpec((_CHUNK, 1), lambda k: (k, 0)),
        out_shape=jax.ShapeDtypeStruct((_G, 1), f32),
    )(s_col, s_row)

    rank_i = rank.reshape(_G).astype(jnp.int32)
    pay_flat = pay.reshape(_G, _FPAD)

    sc_select = pl.kernel(
        _sc_select_body,
        out_type=jax.ShapeDtypeStruct((_NOUT + _NW, _FPAD), f32),
        mesh=plsc.VectorSubcoreMesh(core_axis_name="c", subcore_axis_name="s",
                                    num_cores=2, num_subcores=16),
        compiler_params=pltpu.CompilerParams(use_tc_tiling_on_sc=False),
        scratch_types=[
            pltpu.VMEM((_PERW,), jnp.int32),
            pltpu.VMEM((16, _FPAD), f32),
            pltpu.SemaphoreType.DMA,
        ],
    )
    out = sc_select(rank_i, pay_flat)[:_TOPK]

    return (out[:, 0:4], out[:, 16], out[:, 17].astype(jnp.int32),
            out[:, 4:16])


# speedup vs baseline: 1.0033x; 1.0033x over previous
import jax
import jax.numpy as jnp
from jax.experimental import pallas as pl
from jax.experimental.pallas import tpu as pltpu
from jax.experimental.pallas import tpu_sc as plsc

_NCLS = 8
_N = 1000
_NPAD = 1024
_G = _NCLS * _NPAD
_CHUNK = 128
_TOPK = 100
_NOUT = 128
_NEG = -1e30
_BIG = 99999.0
_KEEP = 11.0
_FPAD = 32
_NW = 32
_PERW = _G // _NW


def _fiota(shape, axis):
    return jax.lax.broadcasted_iota(jnp.int32, shape, axis).astype(jnp.float32)


def _class_body(acol_ref, arow_ref, p_ref, pay_ref, scc_ref, scr_ref):
    A_c = acol_ref[0]
    A_r = arow_ref[0]
    P = p_ref[0]

    x1c, y1c, x2c, y2c = A_c[0:1, :], A_c[1:2, :], A_c[2:3, :], A_c[3:4, :]
    clsc, cfc = A_c[4:5, :], A_c[5:6, :]
    x1r, y1r, x2r, y2r = A_r[:, 0:1], A_r[:, 1:2], A_r[:, 2:3], A_r[:, 3:4]
    clsr, cfr, phr = A_r[:, 4:5], A_r[:, 5:6], A_r[:, 6:7]

    mx1 = jnp.maximum(x1r, x1c)
    my1 = jnp.maximum(y1r, y1c)
    mx2 = jnp.minimum(x2r, x2c)
    my2 = jnp.minimum(y2r, y2c)
    wid = mx2 - mx1 + 1.0
    hei = my2 - my1 + 1.0
    inter = wid * hei
    area_r = (x2r - x1r + 1.0) * (y2r - y1r + 1.0)
    area_c = (x2c - x1c + 1.0) * (y2c - y1c + 1.0)
    union = area_r + area_c - inter
    ov = jnp.where(union == 0.0, 0.0, inter / jnp.where(union == 0.0, 1.0, union))
    ov = jnp.where(wid <= 0.0, 0.0, ov)
    ov = jnp.where(hei <= 0.0, 0.0, ov)

    validr = clsr > 0.5
    validc = clsc > 0.5
    cond = (ov > 0.5) & validr & validc
    colio = _fiota((_NPAD, _NPAD), 1)
    rowio = _fiota((_NPAD, _NPAD), 0)

    jstar_r = jnp.min(jnp.where(cond, colio, _BIG), axis=1, keepdims=True)
    jstar_c = jnp.min(jnp.where(cond, rowio, _BIG), axis=0, keepdims=True)
    ovs_r = jnp.max(jnp.where(colio == jstar_r, ov, -1.0), axis=1, keepdims=True)
    ovs_c = jnp.max(jnp.where(rowio == jstar_c, ov, -1.0), axis=0, keepdims=True)
    key_r = (1.0 - ovs_r) * cfr
    key_c = (1.0 - ovs_c) * cfc
    mem_r = (jstar_r < _BIG) & (key_r != 0.0)
    mem_c = (jstar_c < _BIG) & (key_c != 0.0)

    samej = jstar_c == jstar_r
    less_cr = (key_c < key_r) | ((key_c == key_r) & (colio < rowio))
    rank_r = jnp.sum(jnp.where(samej & mem_c & less_cr, 1.0, 0.0), axis=1, keepdims=True)
    less_rc = (key_r < key_c) | ((key_r == key_c) & (rowio < colio))
    rank_c = jnp.sum(jnp.where(samej & mem_r & less_rc, 1.0, 0.0), axis=0, keepdims=True)
    sel_r = mem_r & (rank_r < _KEEP)
    sel_c = mem_c & (rank_c < _KEEP)

    W = jnp.where((rowio == jstar_c) & sel_c, 1.0, 0.0)
    Wt = jnp.where((colio == jstar_r) & sel_r, 1.0, 0.0)
    sums = jnp.dot(W, P, preferred_element_type=jnp.float32)
    d_r = jnp.sum(W, axis=1, keepdims=True)
    d_c = jnp.sum(Wt, axis=0, keepdims=True)
    den = jnp.where(d_r == 0.0, 1.0, d_r)
    zero = d_r == 0.0
    poses_o = jnp.where(zero, 0.0, sums[:, 0:12] / den)
    boxes_o = jnp.where(zero, 0.0, sums[:, 12:16] / den)

    anyv = jnp.max(jnp.where(validc, 1.0, 0.0))
    riota = _fiota((_NPAD, 1), 0)
    ciota = _fiota((1, _NPAD), 1)
    real_r = riota < float(_N)
    keep_r = (d_r > 0.0) & validr & real_r
    keep_c = (d_c > 0.0) & validc & (ciota < float(_N))
    sc_r = jnp.where(keep_r, clsr, _NEG)
    sc_c = jnp.where(keep_c, clsc, _NEG)
    ph_on = anyv == 0.0
    sc_r = jnp.where((riota == float(_N)) & ph_on, phr, sc_r)
    sc_c = jnp.where((ciota == float(_N)) & ph_on, A_c[6:7, :], sc_c)
    scr_ref[0] = sc_r
    scc_ref[0] = sc_c

    cidf = jnp.full((_NPAD, 1), pl.program_id(0), jnp.float32)
    pay_box = jnp.where(keep_r, boxes_o, -1.0)
    pay_pos = jnp.where(keep_r, poses_o, -1.0)
    pay_scr = jnp.where(sc_r == _NEG, -1.0, sc_r)
    pay_lbl = jnp.where(keep_r, cidf, -1.0)
    pay_ref[0] = jnp.concatenate(
        [pay_box, pay_pos, pay_scr, pay_lbl,
         jnp.zeros((_NPAD, _FPAD - 18), jnp.float32)], axis=1)


def _rank_body(scol_ref, srow_ref, rank_ref):
    sc = scol_ref[0:1, :]
    sr = srow_ref[...]
    k = pl.program_id(0)
    rio = _fiota((_CHUNK, 1), 0) + k.astype(jnp.float32) * float(_CHUNK)
    cio = _fiota((_CHUNK, _G), 1)
    ahead = (sc > sr) | ((sc == sr) & (cio < rio))
    rank_ref[...] = jnp.sum(jnp.where(ahead, 1.0, 0.0), axis=1, keepdims=True)


def _sc_select_body(rank_hbm, pay_hbm, out_hbm, rank_v, rows_v, sem):
    cidx = jax.lax.axis_index("c")
    sidx = jax.lax.axis_index("s")
    wid = sidx * 2 + cidx
    base = wid * _PERW
    pltpu.sync_copy(rank_hbm.at[pl.ds(base, _PERW)], rank_v)
    pltpu.sync_copy(pay_hbm.at[pl.ds(base, _PERW)], rows_v)
    dump = _NOUT + wid

    copies = []
    for ch in range(_PERW // 16):
        rk = rank_v[pl.ds(ch * 16, 16)]
        slot = jnp.where(rk < _NOUT, rk, dump)
        copies.append(
            pltpu.async_copy(rows_v.at[pl.ds(ch * 16, 16)],
                             out_hbm.at[slot], sem))
    for c in copies:
        c.wait()


def kernel(boxes3D, boxes, classification, poses, confidence):
    del boxes3D
    f32 = jnp.float32
    bx = boxes.reshape(_N, _NCLS, 4).astype(f32)
    cls2 = classification.reshape(_N, _NCLS).astype(f32)
    cf2 = confidence.reshape(_N, _NCLS).astype(f32)
    ps2 = poses.reshape(_N, _NCLS, 12).astype(f32)
    ph_score = cls2[-1, -1]

    feat = jnp.stack(
        [bx[..., 0], bx[..., 1], bx[..., 2], bx[..., 3], cls2, cf2,
         jnp.broadcast_to(ph_score, (_N, _NCLS)), jnp.zeros((_N, _NCLS), f32)],
        axis=-1)
    feat = jnp.pad(feat, ((0, _NPAD - _N), (0, 0), (0, 0)))
    a_row = feat.transpose(1, 0, 2)
    a_col = feat.transpose(1, 2, 0)
    pmat = jnp.concatenate([ps2, bx], axis=-1)
    pmat = jnp.pad(pmat, ((0, _NPAD - _N), (0, 0), (0, 0))).transpose(1, 0, 2)

    pay, sc_c, sc_r = pl.pallas_call(
        _class_body,
        grid=(_NCLS,),
        in_specs=[
            pl.BlockSpec((1, 8, _NPAD), lambda c: (c, 0, 0)),
            pl.BlockSpec((1, _NPAD, 8), lambda c: (c, 0, 0)),
            pl.BlockSpec((1, _NPAD, 16), lambda c: (c, 0, 0)),
        ],
        out_specs=[
            pl.BlockSpec((1, _NPAD, _FPAD), lambda c: (c, 0, 0)),
            pl.BlockSpec((1, 1, _NPAD), lambda c: (c, 0, 0)),
            pl.BlockSpec((1, _NPAD, 1), lambda c: (c, 0, 0)),
        ],
        out_shape=[
            jax.ShapeDtypeStruct((_NCLS, _NPAD, _FPAD), f32),
            jax.ShapeDtypeStruct((_NCLS, 1, _NPAD), f32),
            jax.ShapeDtypeStruct((_NCLS, _NPAD, 1), f32),
        ],
    )(a_col, a_row, pmat)

    s_col = sc_c.reshape(1, _G)
    s_row = sc_r.reshape(_G, 1)

    rank = pl.pallas_call(
        _rank_body,
        grid=(_G // _CHUNK,),
        in_specs=[
            pl.BlockSpec((1, _G), lambda k: (0, 0)),
            pl.BlockSpec((_CHUNK, 1), lambda k: (k, 0)),
        ],
        out_specs=pl.BlockSpec((_CHUNK, 1), lambda k: (k, 0)),
        out_shape=jax.ShapeDtypeStruct((_G, 1), f32),
    )(s_col, s_row)

    rank_i = rank.reshape(_G).astype(jnp.int32)
    pay_flat = pay.reshape(_G, _FPAD)

    sc_select = pl.kernel(
        _sc_select_body,
        out_type=jax.ShapeDtypeStruct((_NOUT + _NW, _FPAD), f32),
        mesh=plsc.VectorSubcoreMesh(core_axis_name="c", subcore_axis_name="s",
                                    num_cores=2, num_subcores=16),
        compiler_params=pltpu.CompilerParams(use_tc_tiling_on_sc=False),
        scratch_types=[
            pltpu.VMEM((_PERW,), jnp.int32),
            pltpu.VMEM((_PERW, _FPAD), f32),
            pltpu.SemaphoreType.DMA,
        ],
    )
    out = sc_select(rank_i, pay_flat)[:_TOPK]

    return (out[:, 0:4], out[:, 16], out[:, 17].astype(jnp.int32),
            out[:, 4:16])
